# baseline (speedup 1.0000x reference)
import functools

import jax
import jax.numpy as jnp
from jax import lax
from jax.experimental import pallas as pl
from jax.experimental.pallas import tpu as pltpu
from jax.experimental.pallas import tpu_sc as plsc

B = 16384
F = 26
V = 100000
D = 16
NUM = 13
OUT_W = F * D + NUM

NC = 2
NS = 16
NW = NC * NS

B_PER_W = B // NW
NB = 64
NCHUNK = B_PER_W // NB
ROWS = NB * F
NIDX = 128
NGATHER = ROWS // NIDX
OV_LEN = NB * OUT_W


def _sc_body(xcat_hbm, xnum_hbm, table_hbm, out_hbm, gidx_v, g_v, n_v, o_v, sem):
    cid = lax.axis_index("c")
    sid = lax.axis_index("s")
    wid = sid * NC + cid

    iota = lax.iota(jnp.int32, 16)

    def chunk_body(c, carry):
        g = wid * NCHUNK + c

        pltpu.sync_copy(xcat_hbm.at[pl.ds(g * ROWS, ROWS)], gidx_v)

        def off_body(j, carry2):
            pos = j * 16
            f = lax.rem(pos + iota, F)
            gidx_v[pl.ds(pos, 16)] = gidx_v[pl.ds(pos, 16)] + f * V
            return carry2

        lax.fori_loop(0, ROWS // 16, off_body, 0)

        descs = [
            pltpu.async_copy(
                table_hbm.at[gidx_v.at[pl.ds(j * NIDX, NIDX)]],
                g_v.at[pl.ds(j * NIDX, NIDX)],
                sem,
            )
            for j in range(NGATHER)
        ]
        for d in descs:
            d.wait()

        pltpu.sync_copy(xnum_hbm.at[pl.ds(g * NB * NUM, NB * NUM)], n_v.at[pl.ds(0, NB * NUM)])

        def row_body(i, carry2):
            ob = i * OUT_W
            o_v[pl.ds(ob + F * D, 16)] = n_v[pl.ds(i * NUM, 16)]
            for k in range(F):
                o_v[pl.ds(ob + k * D, 16)] = g_v[i * F + k]
            return carry2

        lax.fori_loop(0, NB, row_body, 0)

        pltpu.sync_copy(o_v.at[pl.ds(0, OV_LEN)], out_hbm.at[pl.ds(g * OV_LEN, OV_LEN)])
        return carry

    lax.fori_loop(0, NCHUNK, chunk_body, 0)


_sc_call = pl.kernel(
    _sc_body,
    out_type=jax.ShapeDtypeStruct((B * OUT_W,), jnp.float32),
    mesh=plsc.VectorSubcoreMesh(core_axis_name="c", subcore_axis_name="s"),
    compiler_params=pltpu.CompilerParams(use_tc_tiling_on_sc=False),
    scratch_types=[
        pltpu.VMEM((ROWS,), jnp.int32),
        pltpu.VMEM((ROWS, D), jnp.float32),
        pltpu.VMEM((NB * NUM + 16,), jnp.float32),
        pltpu.VMEM((OV_LEN + 16,), jnp.float32),
        pltpu.SemaphoreType.DMA,
    ],
)


@jax.jit
def kernel(x_categorical, x_numerical, tables):
    xcat = x_categorical.reshape(B * F)
    xnum = x_numerical.reshape(B * NUM)
    tab = tables.reshape(F * V, D)
    out = _sc_call(xcat, xnum, tab)
    return out.reshape(B, OUT_W)


# speedup vs baseline: 5.1997x; 5.1997x over previous
import functools

import jax
import jax.numpy as jnp
from jax import lax
from jax.experimental import pallas as pl
from jax.experimental.pallas import tpu as pltpu
from jax.experimental.pallas import tpu_sc as plsc

B = 16384
F = 26
V = 100000
D = 16
NUM = 13
C = F * D
OUT_H = C + NUM

HB = B // 2


def _sc_body(tab, xcat, xnum, out, sp_out, sp_idx, t_row, idx_v, out_v, sem):
    cid = lax.axis_index("c")
    sid = lax.axis_index("s")

    for g in range(F // 2):
        f = 2 * g + cid

        @pl.when(sid == 1)
        def _():
            pltpu.sync_copy(xcat.at[pl.ds(2 * g, 2), :], sp_idx)

        pltpu.sync_copy(tab.at[f, sid, :], t_row)

        for h in range(2):
            plsc.subcore_barrier()

            pltpu.sync_copy(sp_idx.at[cid, pl.ds(h * HB, HB)], idx_v)

            def gather_body(j, carry):
                iv = idx_v[pl.ds(j * 16, 16)]
                out_v[pl.ds(j * 16, 16)] = plsc.load_gather(t_row, [iv])
                return carry

            lax.fori_loop(0, HB // 16, gather_body, 0)

            pltpu.sync_copy(out_v, sp_out.at[sid])
            plsc.subcore_barrier()

            @pl.when(sid == 0)
            def _():
                pltpu.sync_copy(sp_out, out.at[pl.ds(f * D, D), pl.ds(h * HB, HB)])

        plsc.subcore_barrier()

    for h in range(2):
        @pl.when((sid == 0) & (cid == 0))
        def _():
            pltpu.sync_copy(
                xnum.at[pl.ds(0, 8), pl.ds(h * HB, HB)], sp_out.at[pl.ds(0, 8)]
            )
            pltpu.sync_copy(
                sp_out.at[pl.ds(0, 8)], out.at[pl.ds(C, 8), pl.ds(h * HB, HB)]
            )

        @pl.when((sid == 0) & (cid == 1))
        def _():
            pltpu.sync_copy(
                xnum.at[pl.ds(8, 5), pl.ds(h * HB, HB)], sp_out.at[pl.ds(0, 5)]
            )
            pltpu.sync_copy(
                sp_out.at[pl.ds(0, 5)], out.at[pl.ds(C + 8, 5), pl.ds(h * HB, HB)]
            )


_sc_call = pl.kernel(
    _sc_body,
    out_type=jax.ShapeDtypeStruct((OUT_H, B), jnp.float32),
    mesh=plsc.VectorSubcoreMesh(core_axis_name="c", subcore_axis_name="s"),
    compiler_params=pltpu.CompilerParams(
        use_tc_tiling_on_sc=True, needs_layout_passes=False
    ),
    scratch_types=[
        pltpu.VMEM_SHARED((D, HB), jnp.float32),
        pltpu.VMEM_SHARED((2, B), jnp.int32),
        pltpu.VMEM((V,), jnp.float32),
        pltpu.VMEM((HB,), jnp.int32),
        pltpu.VMEM((HB,), jnp.float32),
        pltpu.SemaphoreType.DMA,
    ],
)


@jax.jit
def kernel(x_categorical, x_numerical, tables):
    tab_t = jnp.transpose(tables, (0, 2, 1))
    xcat_t = jnp.transpose(x_categorical, (1, 0))
    xnum_t = jnp.transpose(x_numerical, (1, 0))
    out_t = _sc_call(tab_t, xcat_t, xnum_t)
    return jnp.transpose(out_t, (1, 0))


# speedup vs baseline: 6.7261x; 1.2936x over previous
import functools

import jax
import jax.numpy as jnp
from jax import lax
from jax.experimental import pallas as pl
from jax.experimental.pallas import tpu as pltpu
from jax.experimental.pallas import tpu_sc as plsc

B = 16384
F = 26
V = 100000
D = 16
NUM = 13
C = F * D
OUT_H = C + NUM

QB = B // 4
UNROLL = 4


def _sc_body(
    tab, xcat, xnum, out, sp_out, sp_idx, t_row, idx_v, out_v, sem_in, sem_out
):
    cid = lax.axis_index("c")
    sid = lax.axis_index("s")

    pltpu.async_copy(tab.at[cid, sid, :], t_row, sem_in)

    @pl.when(sid == 1)
    def _():
        pltpu.sync_copy(xcat.at[pl.ds(0, 2), :], sp_idx)

    out_desc = None
    for g in range(F // 2):
        f = 2 * g + cid

        pltpu.make_async_copy(tab.at[cid, sid, :], t_row, sem_in).wait()
        if out_desc is not None:
            @pl.when(sid == 0)
            def _():
                pltpu.make_async_copy(
                    sp_out, out.at[pl.ds(0, D), :], sem_out
                ).wait()

        plsc.subcore_barrier()

        for q in range(4):
            pltpu.sync_copy(sp_idx.at[cid, pl.ds(q * QB, QB)], idx_v)

            def gather_body(j, carry):
                for u in range(UNROLL):
                    o = j * (16 * UNROLL) + u * 16
                    iv = idx_v[pl.ds(o, 16)]
                    out_v[pl.ds(o, 16)] = plsc.load_gather(t_row, [iv])
                return carry

            lax.fori_loop(0, QB // (16 * UNROLL), gather_body, 0)

            pltpu.sync_copy(out_v, sp_out.at[sid, pl.ds(q * QB, QB)])

        if g + 1 < F // 2:
            pltpu.async_copy(tab.at[2 * (g + 1) + cid, sid, :], t_row, sem_in)

        plsc.subcore_barrier()

        if g + 1 < F // 2:
            @pl.when(sid == 1)
            def _():
                pltpu.sync_copy(xcat.at[pl.ds(2 * (g + 1), 2), :], sp_idx)

        @pl.when(sid == 0)
        def _():
            pltpu.async_copy(sp_out, out.at[pl.ds(f * D, D), :], sem_out)
        out_desc = True

    @pl.when(sid == 0)
    def _():
        pltpu.make_async_copy(sp_out, out.at[pl.ds(0, D), :], sem_out).wait()

    @pl.when((sid == 0) & (cid == 0))
    def _():
        pltpu.sync_copy(xnum.at[pl.ds(0, 8), :], sp_out.at[pl.ds(0, 8)])
        pltpu.sync_copy(sp_out.at[pl.ds(0, 8)], out.at[pl.ds(C, 8), :])

    @pl.when((sid == 0) & (cid == 1))
    def _():
        pltpu.sync_copy(xnum.at[pl.ds(8, 5), :], sp_out.at[pl.ds(0, 5)])
        pltpu.sync_copy(sp_out.at[pl.ds(0, 5)], out.at[pl.ds(C + 8, 5), :])


_sc_call = pl.kernel(
    _sc_body,
    out_type=jax.ShapeDtypeStruct((OUT_H, B), jnp.float32),
    mesh=plsc.VectorSubcoreMesh(core_axis_name="c", subcore_axis_name="s"),
    compiler_params=pltpu.CompilerParams(
        use_tc_tiling_on_sc=True, needs_layout_passes=False
    ),
    scratch_types=[
        pltpu.VMEM_SHARED((D, B), jnp.float32),
        pltpu.VMEM_SHARED((2, B), jnp.int32),
        pltpu.VMEM((V,), jnp.float32),
        pltpu.VMEM((QB,), jnp.int32),
        pltpu.VMEM((QB,), jnp.float32),
        pltpu.SemaphoreType.DMA,
        pltpu.SemaphoreType.DMA,
    ],
)


@jax.jit
def kernel(x_categorical, x_numerical, tables):
    tab_t = jnp.transpose(tables, (0, 2, 1))
    xcat_t = jnp.transpose(x_categorical, (1, 0))
    xnum_t = jnp.transpose(x_numerical, (1, 0))
    out_t = _sc_call(tab_t, xcat_t, xnum_t)
    return jnp.transpose(out_t, (1, 0))
